# baseline (speedup 1.0000x reference)
import jax
import jax.numpy as jnp
from jax import lax
from jax.experimental import pallas as pl

_B = 256
_E = 4096
_N = 256
_TD = 128

_BB = 8


def _tc_body(adj_ref, wa_ref, wb_ref, len_ref, out_ref):
    adj = adj_ref[...].astype(jnp.bfloat16)
    acc = jnp.dot(adj, wa_ref[...], preferred_element_type=jnp.float32)
    wb = wb_ref[...]
    lens = len_ref[...]
    niota = lax.broadcasted_iota(jnp.int32, (_N, _TD), 0)
    for k in range(_BB):
        valid = niota < lens[k:k + 1, :]
        blk = acc[k * _N:(k + 1) * _N, :] + wb
        out_ref[pl.ds(k * _N, _N), :] = jnp.where(valid, blk, 0.0)


def _tc_embed(adj2, wadjT, wseb, len_bcast):
    return pl.pallas_call(
        _tc_body,
        grid=(_B // _BB,),
        in_specs=[
            pl.BlockSpec((_BB * _N, _N), lambda i: (i, 0)),
            pl.BlockSpec((_N, _TD), lambda i: (0, 0)),
            pl.BlockSpec((_N, _TD), lambda i: (0, 0)),
            pl.BlockSpec((_BB, _TD), lambda i: (i, 0)),
        ],
        out_specs=pl.BlockSpec((_BB * _N, _TD), lambda i: (i, 0)),
        out_shape=jax.ShapeDtypeStruct((_B * _N, _TD), jnp.float32),
    )(adj2, wadjT, wseb, len_bcast)


def kernel(edge_src, edge_dst, edge_wgt, lengths, W, b):
    bidx = jnp.arange(_B, dtype=jnp.int32)[:, None]
    adj = jnp.zeros((_B, _N, _N), dtype=jnp.float32)
    adj = adj.at[bidx, edge_src, edge_dst].set(edge_wgt)
    adj = adj.at[bidx, edge_dst, edge_src].set(edge_wgt)
    adj2 = adj.reshape(_B * _N, _N)
    wadjT = W[:, :_N].T.astype(jnp.bfloat16)
    wseb = W[:, _N:].T + b[None, :]
    len_bcast = jnp.broadcast_to(lengths[:, None], (_B, _TD))
    out = _tc_embed(adj2, wadjT, wseb, len_bcast)
    return out.reshape(_B, _N, _TD)


# speedup vs baseline: 1.0031x; 1.0031x over previous
import jax
import jax.numpy as jnp
from jax import lax
from jax.experimental import pallas as pl
from jax.experimental.pallas import tpu as pltpu

_B = 256
_E = 4096
_N = 256
_TD = 128

_BB = 8
_FR = _BB * _N * _N // 128


def _tc_body(adj_ref, wa_ref, wb_ref, len_ref, out_ref, g_ref):
    adj = adj_ref[...].astype(jnp.bfloat16)
    g = jnp.dot(adj, wa_ref[...], preferred_element_type=jnp.float32)
    g_ref[...] = g.reshape(_FR // 2, 2, 2 * _TD)
    even = g_ref[:, 0, 0:_TD]
    odd = g_ref[:, 1, _TD:2 * _TD]
    acc = even + odd
    wb = wb_ref[...]
    lens = len_ref[...]
    niota = lax.broadcasted_iota(jnp.int32, (_N, _TD), 0)
    for k in range(_BB):
        valid = niota < lens[k:k + 1, :]
        blk = acc[k * _N:(k + 1) * _N, :] + wb
        out_ref[pl.ds(k * _N, _N), :] = jnp.where(valid, blk, 0.0)


def _tc_embed(adj_flat, wa2, wseb, len_bcast):
    return pl.pallas_call(
        _tc_body,
        grid=(_B // _BB,),
        in_specs=[
            pl.BlockSpec((_FR, 128), lambda i: (i, 0)),
            pl.BlockSpec((128, 2 * _TD), lambda i: (0, 0)),
            pl.BlockSpec((_N, _TD), lambda i: (0, 0)),
            pl.BlockSpec((_BB, _TD), lambda i: (i, 0)),
        ],
        out_specs=pl.BlockSpec((_BB * _N, _TD), lambda i: (i, 0)),
        out_shape=jax.ShapeDtypeStruct((_B * _N, _TD), jnp.float32),
        scratch_shapes=[pltpu.VMEM((_FR // 2, 2, 2 * _TD), jnp.float32)],
    )(adj_flat, wa2, wseb, len_bcast)


def kernel(edge_src, edge_dst, edge_wgt, lengths, W, b):
    bidx = jnp.arange(_B, dtype=jnp.int32)[:, None]
    adj = jnp.zeros((_B, _N, _N), dtype=jnp.float32)
    adj = adj.at[bidx, edge_src, edge_dst].set(edge_wgt)
    adj = adj.at[bidx, edge_dst, edge_src].set(edge_wgt)
    adj_flat = adj.reshape(_B * _N * _N // 128, 128)
    waT = W[:, :_N].T
    wa2 = jnp.concatenate([waT[0:128, :], waT[128:_N, :]], axis=1)
    wa2 = wa2.astype(jnp.bfloat16)
    wseb = W[:, _N:].T + b[None, :]
    len_bcast = jnp.broadcast_to(lengths[:, None], (_B, _TD))
    out = _tc_embed(adj_flat, wa2, wseb, len_bcast)
    return out.reshape(_B, _N, _TD)


# speedup vs baseline: 1.0050x; 1.0020x over previous
import jax
import jax.numpy as jnp
from jax import lax
from jax.experimental import pallas as pl
from jax.experimental.pallas import tpu as pltpu

_B = 256
_E = 4096
_N = 256
_TD = 128

_BB = 8
_FR = _BB * _N * _N // 128


def _tc_body(adj_ref, wa_ref, wb_ref, len_ref, out_ref):
    adj = adj_ref[...].astype(jnp.bfloat16)
    g = jnp.dot(adj, wa_ref[...], preferred_element_type=jnp.float32)
    g3 = g.reshape(_FR // 2, 2, 2 * _TD)
    acc = g3[:, 0, 0:_TD] + g3[:, 1, _TD:2 * _TD]
    wb = wb_ref[...]
    lens = len_ref[...]
    niota = lax.broadcasted_iota(jnp.int32, (_N, _TD), 0)
    for k in range(_BB):
        valid = niota < lens[k:k + 1, :]
        blk = acc[k * _N:(k + 1) * _N, :] + wb
        out_ref[pl.ds(k * _N, _N), :] = jnp.where(valid, blk, 0.0)


def _tc_embed(adj_flat, wa2, wseb, len_bcast):
    return pl.pallas_call(
        _tc_body,
        grid=(_B // _BB,),
        in_specs=[
            pl.BlockSpec((_FR, 128), lambda i: (i, 0)),
            pl.BlockSpec((128, 2 * _TD), lambda i: (0, 0)),
            pl.BlockSpec((_N, _TD), lambda i: (0, 0)),
            pl.BlockSpec((_BB, _TD), lambda i: (i, 0)),
        ],
        out_specs=pl.BlockSpec((_BB * _N, _TD), lambda i: (i, 0)),
        out_shape=jax.ShapeDtypeStruct((_B * _N, _TD), jnp.float32),
    )(adj_flat, wa2, wseb, len_bcast)


def kernel(edge_src, edge_dst, edge_wgt, lengths, W, b):
    bidx = jnp.arange(_B, dtype=jnp.int32)[:, None]
    adj = jnp.zeros((_B, _N, _N), dtype=jnp.float32)
    adj = adj.at[bidx, edge_src, edge_dst].set(edge_wgt)
    adj = adj.at[bidx, edge_dst, edge_src].set(edge_wgt)
    adj_flat = adj.reshape(_B * _N * _N // 128, 128)
    waT = W[:, :_N].T
    wa2 = jnp.concatenate([waT[0:128, :], waT[128:_N, :]], axis=1)
    wa2 = wa2.astype(jnp.bfloat16)
    wseb = W[:, _N:].T + b[None, :]
    len_bcast = jnp.broadcast_to(lengths[:, None], (_B, _TD))
    out = _tc_embed(adj_flat, wa2, wseb, len_bcast)
    return out.reshape(_B, _N, _TD)


# speedup vs baseline: 1.0054x; 1.0004x over previous
import jax
import jax.numpy as jnp
from jax import lax
from jax.experimental import pallas as pl
from jax.experimental.pallas import tpu as pltpu

_B = 256
_E = 4096
_N = 256
_TD = 128

_BB = 16
_FR = _BB * _N * _N // 128


def _tc_body(adj_ref, wa_ref, wb_ref, len_ref, out_ref):
    adj = adj_ref[...].astype(jnp.bfloat16)
    g = jnp.dot(adj, wa_ref[...], preferred_element_type=jnp.float32)
    g3 = g.reshape(_FR // 2, 2, 2 * _TD)
    acc = g3[:, 0, 0:_TD] + g3[:, 1, _TD:2 * _TD]
    wb = wb_ref[...]
    lens = len_ref[...]
    niota = lax.broadcasted_iota(jnp.int32, (_N, _TD), 0)
    for k in range(_BB):
        valid = niota < lens[k:k + 1, :]
        blk = acc[k * _N:(k + 1) * _N, :] + wb
        out_ref[pl.ds(k * _N, _N), :] = jnp.where(valid, blk, 0.0)


def _tc_embed(adj_flat, wa2, wseb, len_bcast):
    return pl.pallas_call(
        _tc_body,
        grid=(_B // _BB,),
        in_specs=[
            pl.BlockSpec((_FR, 128), lambda i: (i, 0)),
            pl.BlockSpec((128, 2 * _TD), lambda i: (0, 0)),
            pl.BlockSpec((_N, _TD), lambda i: (0, 0)),
            pl.BlockSpec((_BB, _TD), lambda i: (i, 0)),
        ],
        out_specs=pl.BlockSpec((_BB * _N, _TD), lambda i: (i, 0)),
        out_shape=jax.ShapeDtypeStruct((_B * _N, _TD), jnp.float32),
    )(adj_flat, wa2, wseb, len_bcast)


def kernel(edge_src, edge_dst, edge_wgt, lengths, W, b):
    bidx = jnp.arange(_B, dtype=jnp.int32)[:, None]
    adj = jnp.zeros((_B, _N, _N), dtype=jnp.float32)
    adj = adj.at[bidx, edge_src, edge_dst].set(edge_wgt)
    adj = adj.at[bidx, edge_dst, edge_src].set(edge_wgt)
    adj_flat = adj.reshape(_B * _N * _N // 128, 128)
    waT = W[:, :_N].T
    wa2 = jnp.concatenate([waT[0:128, :], waT[128:_N, :]], axis=1)
    wa2 = wa2.astype(jnp.bfloat16)
    wseb = W[:, _N:].T + b[None, :]
    len_bcast = jnp.broadcast_to(lengths[:, None], (_B, _TD))
    out = _tc_embed(adj_flat, wa2, wseb, len_bcast)
    return out.reshape(_B, _N, _TD)
